# baseline (speedup 1.0000x reference)
import functools
import math

import jax
import jax.numpy as jnp
import numpy as np
from jax import lax
from jax.experimental import pallas as pl
from jax.experimental.pallas import tpu as pltpu
from jax.experimental.pallas import tpu_sc as plsc

PAD = 0
V = 32000
N = 4096
EPS = 0.1 / (V - 2)
A = 1.0 - 0.1 - EPS
C = 0.9 * math.log(0.9) + 0.1 * math.log(EPS)

EPS32 = np.float32(EPS)
A32 = np.float32(A)
C32 = np.float32(C)


_info = plsc.get_sparse_core_info()
NC, NS, L = _info.num_cores, _info.num_subcores, _info.num_lanes
NW = NC * NS
RPW = N // NW

_mesh = plsc.VectorSubcoreMesh(core_axis_name="c", subcore_axis_name="s")


@functools.partial(
    pl.kernel,
    out_type=[jax.ShapeDtypeStruct((N,), jnp.float32),
              jax.ShapeDtypeStruct((N,), jnp.float32)],
    mesh=_mesh,
    scratch_types=[
        pltpu.VMEM((RPW,), jnp.int32),
        pltpu.VMEM((RPW,), jnp.int32),
        pltpu.VMEM((RPW,), jnp.int32),
        pltpu.VMEM((RPW,), jnp.float32),
        pltpu.VMEM((RPW,), jnp.float32),
        pltpu.VMEM((RPW,), jnp.float32),
        pltpu.VMEM((RPW,), jnp.float32),
        pltpu.SemaphoreType.DMA,
    ],
)
def _sc_gather(lp_hbm, trg_hbm, s_hbm, m_hbm,
               t_v, gi_v, zi_v, g_v, z_v, s_v, m_v, sem):
    wid = lax.axis_index("s") * NC + lax.axis_index("c")
    base = wid * RPW
    pltpu.sync_copy(trg_hbm.at[pl.ds(base, RPW)], t_v)
    for k in range(RPW // L):
        t16 = t_v[pl.ds(k * L, L)]
        rows16 = (base + k * L) + lax.broadcasted_iota(jnp.int32, (L,), 0)
        gi_v[pl.ds(k * L, L)] = rows16 * V + t16
        zi_v[pl.ds(k * L, L)] = rows16 * V
    pltpu.async_copy(lp_hbm.at[gi_v], g_v, sem).wait()
    pltpu.async_copy(lp_hbm.at[zi_v], z_v, sem).wait()
    for k in range(RPW // L):
        sl = pl.ds(k * L, L)
        msk = t_v[sl] != PAD
        s_v[sl] = jnp.where(msk, C32 - A32 * g_v[sl] + EPS32 * z_v[sl],
                            np.float32(0.0))
        m_v[sl] = jnp.where(msk, np.float32(1.0), np.float32(0.0))
    pltpu.sync_copy(s_v, s_hbm.at[pl.ds(base, RPW)])
    pltpu.sync_copy(m_v, m_hbm.at[pl.ds(base, RPW)])


RB = 64
NRB = N // RB
HALF = NRB // 2
VB = 32000
NACC = 4


def _rowsum(x):
    nsl = VB // 128
    accs = [x[:, k * 128:(k + 1) * 128] for k in range(NACC)]
    for k in range(NACC, nsl):
        accs[k % NACC] = accs[k % NACC] + x[:, k * 128:(k + 1) * 128]
    part = accs[0]
    for k in range(1, NACC):
        part = part + accs[k]
    return jnp.sum(part, axis=1)


def _tc_body(lpa_ref, lpb_ref, sa_ref, ma_ref, sb_ref, mb_ref, out_ref):
    r = pl.program_id(0)
    rs_a = _rowsum(lpa_ref[0])
    rs_b = _rowsum(lpb_ref[0])

    @pl.when(r == 0)
    def _():
        out_ref[0, 0] = np.float32(0.0)

    tot = (jnp.sum(sa_ref[0, 0, :]) - EPS32 * jnp.sum(ma_ref[0, 0, :] * rs_a)
           + jnp.sum(sb_ref[0, 0, :]) - EPS32 * jnp.sum(mb_ref[0, 0, :] * rs_b))
    out_ref[0, 0] += tot


def _tc_reduce(lp3, s3, m3):
    sm_spec_a = pl.BlockSpec((1, 1, RB), lambda r: (r, 0, 0))
    sm_spec_b = pl.BlockSpec((1, 1, RB), lambda r: (r + HALF, 0, 0))
    return pl.pallas_call(
        _tc_body,
        grid=(HALF,),
        in_specs=[
            pl.BlockSpec((1, RB, VB), lambda r: (0, r, 0)),
            pl.BlockSpec((1, RB, VB), lambda r: (1, r, 0)),
            sm_spec_a,
            sm_spec_a,
            sm_spec_b,
            sm_spec_b,
        ],
        out_specs=pl.BlockSpec(
            (1, 1), lambda r: (0, 0), memory_space=pltpu.SMEM),
        out_shape=jax.ShapeDtypeStruct((1, 1), jnp.float32),
    )(lp3, lp3, s3, m3, s3, m3)


def kernel(log_probs, trg):
    lp3 = log_probs.reshape(2, N // 2, V)
    lp_flat = log_probs.reshape(N * V)
    t_flat = trg.reshape(N).astype(jnp.int32)
    s, m = _sc_gather(lp_flat, t_flat)
    out = _tc_reduce(lp3, s.reshape(NRB, 1, RB), m.reshape(NRB, 1, RB))
    return (out.reshape(()),)


# speedup vs baseline: 1.0084x; 1.0084x over previous
import functools
import math

import jax
import jax.numpy as jnp
import numpy as np
from jax import lax
from jax.experimental import pallas as pl
from jax.experimental.pallas import tpu as pltpu
from jax.experimental.pallas import tpu_sc as plsc

PAD = 0
V = 32000
N = 4096
EPS = 0.1 / (V - 2)
A = 1.0 - 0.1 - EPS
C = 0.9 * math.log(0.9) + 0.1 * math.log(EPS)

EPS32 = np.float32(EPS)
A32 = np.float32(A)
C32 = np.float32(C)


_info = plsc.get_sparse_core_info()
NC, NS, L = _info.num_cores, _info.num_subcores, _info.num_lanes
NW = NC * NS
RPW = N // NW

_mesh = plsc.VectorSubcoreMesh(core_axis_name="c", subcore_axis_name="s")


@functools.partial(
    pl.kernel,
    out_type=[jax.ShapeDtypeStruct((N,), jnp.float32),
              jax.ShapeDtypeStruct((N,), jnp.float32)],
    mesh=_mesh,
    scratch_types=[
        pltpu.VMEM((RPW,), jnp.int32),
        pltpu.VMEM((RPW,), jnp.int32),
        pltpu.VMEM((RPW,), jnp.int32),
        pltpu.VMEM((RPW,), jnp.float32),
        pltpu.VMEM((RPW,), jnp.float32),
        pltpu.VMEM((RPW,), jnp.float32),
        pltpu.VMEM((RPW,), jnp.float32),
        pltpu.SemaphoreType.DMA,
    ],
)
def _sc_gather(lp_hbm, trg_hbm, s_hbm, m_hbm,
               t_v, gi_v, zi_v, g_v, z_v, s_v, m_v, sem):
    wid = lax.axis_index("s") * NC + lax.axis_index("c")
    base = wid * RPW
    pltpu.sync_copy(trg_hbm.at[pl.ds(base, RPW)], t_v)
    for k in range(RPW // L):
        t16 = t_v[pl.ds(k * L, L)]
        rows16 = (base + k * L) + lax.broadcasted_iota(jnp.int32, (L,), 0)
        gi_v[pl.ds(k * L, L)] = rows16 * V + t16
        zi_v[pl.ds(k * L, L)] = rows16 * V
    pltpu.async_copy(lp_hbm.at[gi_v], g_v, sem).wait()
    pltpu.async_copy(lp_hbm.at[zi_v], z_v, sem).wait()
    for k in range(RPW // L):
        sl = pl.ds(k * L, L)
        msk = t_v[sl] != PAD
        s_v[sl] = jnp.where(msk, C32 - A32 * g_v[sl] + EPS32 * z_v[sl],
                            np.float32(0.0))
        m_v[sl] = jnp.where(msk, np.float32(1.0), np.float32(0.0))
    pltpu.sync_copy(s_v, s_hbm.at[pl.ds(base, RPW)])
    pltpu.sync_copy(m_v, m_hbm.at[pl.ds(base, RPW)])


RB = 128
NRB = N // RB
VB = 32000
NACC = 4


def _rowsum(x):
    nsl = VB // 128
    accs = [x[:, k * 128:(k + 1) * 128] for k in range(NACC)]
    for k in range(NACC, nsl):
        accs[k % NACC] = accs[k % NACC] + x[:, k * 128:(k + 1) * 128]
    part = accs[0]
    for k in range(1, NACC):
        part = part + accs[k]
    return jnp.sum(part, axis=1)


def _tc_body(lp_ref, rs_ref):
    rs_ref[0, 0, :] = _rowsum(lp_ref[...])


def _tc_rowsums(lp2):
    return pl.pallas_call(
        _tc_body,
        grid=(NRB,),
        in_specs=[pl.BlockSpec((RB, VB), lambda r: (r, 0))],
        out_specs=pl.BlockSpec((1, 1, RB), lambda r: (r, 0, 0)),
        out_shape=jax.ShapeDtypeStruct((NRB, 1, RB), jnp.float32),
    )(lp2)


def _combine_body(rs_ref, s_ref, m_ref, out_ref):
    out_ref[0, 0] = (jnp.sum(s_ref[...])
                     - EPS32 * jnp.sum(m_ref[...] * rs_ref[...]))


def _combine(rs3, s3, m3):
    return pl.pallas_call(
        _combine_body,
        in_specs=[
            pl.BlockSpec((NRB, 1, RB), lambda: (0, 0, 0)),
            pl.BlockSpec((NRB, 1, RB), lambda: (0, 0, 0)),
            pl.BlockSpec((NRB, 1, RB), lambda: (0, 0, 0)),
        ],
        out_specs=pl.BlockSpec(
            (1, 1), lambda: (0, 0), memory_space=pltpu.SMEM),
        out_shape=jax.ShapeDtypeStruct((1, 1), jnp.float32),
    )(rs3, s3, m3)


def kernel(log_probs, trg):
    lp2 = log_probs.reshape(N, V)
    lp_flat = log_probs.reshape(N * V)
    t_flat = trg.reshape(N).astype(jnp.int32)
    s, m = _sc_gather(lp_flat, t_flat)
    rs3 = _tc_rowsums(lp2)
    out = _combine(rs3, s.reshape(NRB, 1, RB), m.reshape(NRB, 1, RB))
    return (out.reshape(()),)
